# baseline (speedup 1.0000x reference)
import functools

import jax
import jax.numpy as jnp
from jax import lax
from jax.experimental import pallas as pl
from jax.experimental.pallas import tpu as pltpu
from jax.experimental.pallas import tpu_sc as plsc

B = 64
PATCH = 16
IN_CHANS = 3
EMBED = 768
NUM_PATCHES = 1024
N_VIS = 256
N_MASK = 768
ROWS = 1 + NUM_PATCHES

NC = 2
NS = 16
NW = NC * NS
RSPAN = NUM_PATCHES // NW
NLANE = EMBED // 16

GRAY = 127.0 / 255.0


def _body(x_hbm, samp_hbm, mask_hbm, w_hbm, b_hbm, out_hbm,
          samp_v, mask_v, code_s, xpos_s, xcol_s, wrow_v, b_v,
          crow_v, zrow_v, hstage, buf_a, buf_b, lbuf, xdummy,
          sem_a, sem_b, sem_g):
    wid = lax.axis_index("s") * NC + lax.axis_index("c")
    r0 = wid * RSPAN

    pltpu.sync_copy(samp_hbm, samp_v)
    pltpu.sync_copy(mask_hbm, mask_v)
    pltpu.sync_copy(w_hbm, wrow_v)
    pltpu.sync_copy(b_hbm.at[pl.ds(0, 16)], b_v)

    def sum_step(i, acc):
        return acc + wrow_v[pl.ds(i * 16, 16)]
    acc = lax.fori_loop(0, NLANE, sum_step, jnp.zeros((16,), jnp.float32))
    tot = acc[0]
    for i in range(1, 16):
        tot = tot + acc[i]
    s = tot * jnp.float32(GRAY) + b_v[...][0]
    vs = jnp.full((16,), s, dtype=jnp.float32)
    vz = jnp.zeros((16,), jnp.float32)

    def init_step(t, _):
        code_s[t] = jnp.int32(-1)
        return 0
    lax.fori_loop(0, ROWS, init_step, 0)

    code_s[0] = jnp.int32(0)

    def samp_step(g, _):
        v = samp_v[pl.ds(g * 16, 16)]
        for i in range(16):
            code_s[v[i]] = g * 16 + i + 1
        return 0
    lax.fori_loop(0, N_VIS // 16, samp_step, 0)

    def mask_step(g, _):
        v = mask_v[pl.ds(g * 16, 16)]
        for i in range(16):
            code_s[v[i]] = jnp.int32(-2)
        return 0
    lax.fori_loop(0, N_MASK // 16, mask_step, 0)

    def row_fill(col, _):
        crow_v[pl.ds(col * 16, 16)] = vs
        zrow_v[pl.ds(col * 16, 16)] = vz
        return 0
    lax.fori_loop(0, NLANE, row_fill, 0)
    pltpu.sync_copy(crow_v, hstage.at[wid, 0])
    pltpu.sync_copy(zrow_v, hstage.at[wid, 1])
    crow_sh = hstage.at[wid, 0]
    zrow_sh = hstage.at[wid, 1]

    def tmpl_desc(ri, c, buf):
        src = jnp.where(c == -2, 0, 1)
        return pltpu.make_async_copy(hstage.at[wid, src], buf.at[0, ri],
                                     sem_g)

    nx = jnp.int32(0)
    for ri in range(RSPAN):
        c = code_s[r0 + ri]
        is_x = c >= 0

        @pl.when(jnp.logical_not(is_x))
        def _(c=c, ri=ri):
            tmpl_desc(ri, c, buf_a).start()
            tmpl_desc(ri, c, buf_b).start()

        nx = nx + jnp.where(is_x, 1, 0)
    for ri in range(RSPAN):
        c = code_s[r0 + ri]

        @pl.when(c < 0)
        def _(c=c, ri=ri):
            tmpl_desc(ri, c, buf_a).wait()
            tmpl_desc(ri, c, buf_b).wait()

    def x_desc(bb, ri, c, buf):
        off = (bb * (1 + N_VIS) + c) * EMBED
        return pltpu.make_async_copy(x_hbm.at[pl.ds(off, EMBED)],
                                     buf.at[0, ri], sem_g)

    def write_desc(bb, buf, sem):
        return pltpu.make_async_copy(
            buf, out_hbm.at[pl.ds(bb, 1), pl.ds(r0, RSPAN), :], sem)

    def do_unit(bb, buf, sem):
        for ri in range(RSPAN):
            c = code_s[r0 + ri]

            @pl.when(c >= 0)
            def _(c=c, ri=ri):
                x_desc(bb, ri, c, buf).start()
        for ri in range(RSPAN):
            c = code_s[r0 + ri]

            @pl.when(c >= 0)
            def _(c=c, ri=ri):
                x_desc(bb, ri, c, buf).wait()
        write_desc(bb, buf, sem).start()

    def b_step(t, _):
        @pl.when(t > 0)
        def _():
            write_desc(2 * t - 2, buf_a, sem_a).wait()
            write_desc(2 * t - 1, buf_b, sem_b).wait()
        do_unit(2 * t, buf_a, sem_a)
        do_unit(2 * t + 1, buf_b, sem_b)
        return 0
    lax.fori_loop(0, B // 2, b_step, 0)
    write_desc(B - 2, buf_a, sem_a).wait()
    write_desc(B - 1, buf_b, sem_b).wait()

    cl = code_s[NUM_PATCHES]

    @pl.when(cl == -2)
    def _():
        pltpu.sync_copy(crow_sh, lbuf.at[0, 0])

    @pl.when(cl == -1)
    def _():
        pltpu.sync_copy(zrow_sh, lbuf.at[0, 0])

    for k in range(2):
        bb = 2 * wid + k

        @pl.when(cl >= 0)
        def _():
            pltpu.sync_copy(
                x_hbm.at[pl.ds((bb * (1 + N_VIS) + cl) * EMBED, EMBED)],
                lbuf.at[0, 0])
        pltpu.sync_copy(
            lbuf, out_hbm.at[pl.ds(bb, 1), pl.ds(NUM_PATCHES, 1), :])


@functools.partial(jax.jit, static_argnames=())
def kernel(x, sample_index, mask_index, W, b):
    x_lin = x.reshape(B * (1 + N_VIS) * EMBED)
    wrow = W[0].reshape(IN_CHANS * PATCH * PATCH)
    run = pl.kernel(
        _body,
        mesh=plsc.VectorSubcoreMesh(core_axis_name="c", subcore_axis_name="s"),
        out_type=jax.ShapeDtypeStruct((B, ROWS, EMBED), jnp.float32),
        compiler_params=pltpu.CompilerParams(use_tc_tiling_on_sc=True),
        scratch_types=[
            pltpu.VMEM((N_VIS,), jnp.int32),
            pltpu.VMEM((N_MASK,), jnp.int32),
            pltpu.SMEM((ROWS,), jnp.int32),
            pltpu.SMEM((RSPAN,), jnp.int32),
            pltpu.SMEM((RSPAN,), jnp.int32),
            pltpu.VMEM((EMBED,), jnp.float32),
            pltpu.VMEM((16,), jnp.float32),
            pltpu.VMEM((EMBED,), jnp.float32),
            pltpu.VMEM((EMBED,), jnp.float32),
            pltpu.HBM((NW, 8, EMBED), jnp.float32),
            pltpu.VMEM((1, RSPAN, EMBED), jnp.float32),
            pltpu.VMEM((1, RSPAN, EMBED), jnp.float32),
            pltpu.VMEM((1, 1, EMBED), jnp.float32),
            pltpu.VMEM((EMBED,), jnp.float32),
            pltpu.SemaphoreType.DMA,
            pltpu.SemaphoreType.DMA,
            pltpu.SemaphoreType.DMA,
        ],
    )
    return run(x_lin, sample_index, mask_index, wrow, b)


# speedup vs baseline: 3.3251x; 3.3251x over previous
import functools

import jax
import jax.numpy as jnp
from jax import lax
from jax.experimental import pallas as pl
from jax.experimental.pallas import tpu as pltpu
from jax.experimental.pallas import tpu_sc as plsc

B = 64
PATCH = 16
IN_CHANS = 3
EMBED = 768
NUM_PATCHES = 1024
N_VIS = 256
N_MASK = 768
ROWS = 1 + NUM_PATCHES

NC = 2
NS = 16
NW = NC * NS
RSPAN = NUM_PATCHES // NW
NLANE = EMBED // 16
HB = B // 2
LAG = 6

GRAY = 127.0 / 255.0


def _body(x_hbm, samp_hbm, mask_hbm, w_hbm, b_hbm, out_hbm,
          samp_v, mask_v, code_s, wrow_v, b_v,
          cbuf, zbuf, xb0, xb1, sem_w, sem_x, sem_g):
    wid = lax.axis_index("s") * NC + lax.axis_index("c")
    r0 = wid * RSPAN

    pltpu.sync_copy(samp_hbm, samp_v)
    pltpu.sync_copy(mask_hbm, mask_v)
    pltpu.sync_copy(w_hbm, wrow_v)
    pltpu.sync_copy(b_hbm.at[pl.ds(0, 16)], b_v)

    def sum_step(i, acc):
        return acc + wrow_v[pl.ds(i * 16, 16)]
    acc = lax.fori_loop(0, NLANE, sum_step, jnp.zeros((16,), jnp.float32))
    tot = acc[0]
    for i in range(1, 16):
        tot = tot + acc[i]
    s = tot * jnp.float32(GRAY) + b_v[...][0]
    vs = jnp.full((16,), s, dtype=jnp.float32)
    vz = jnp.zeros((16,), jnp.float32)

    def init_step(t, _):
        code_s[t] = jnp.int32(-1)
        return 0
    lax.fori_loop(0, ROWS, init_step, 0)

    code_s[0] = jnp.int32(0)

    def samp_step(g, _):
        v = samp_v[pl.ds(g * 16, 16)]
        for i in range(16):
            code_s[v[i]] = g * 16 + i + 1
        return 0
    lax.fori_loop(0, N_VIS // 16, samp_step, 0)

    def mask_step(g, _):
        v = mask_v[pl.ds(g * 16, 16)]
        for i in range(16):
            code_s[v[i]] = jnp.int32(-2)
        return 0
    lax.fori_loop(0, N_MASK // 16, mask_step, 0)

    def fill_row(row, _):
        def fill_col(col, _):
            cbuf[row, pl.ds(col * 16, 16)] = vs
            zbuf[row, pl.ds(col * 16, 16)] = vz
            return 0
        lax.fori_loop(0, NLANE, fill_col, 0)
        return 0
    lax.fori_loop(0, HB, fill_row, 0)

    def wdesc(buf, r, h, sem):
        return pltpu.make_async_copy(
            buf, out_hbm.at[r, pl.ds(h * HB, HB), :], sem)

    def gdesc(cc, h, buf):
        return pltpu.make_async_copy(
            x_hbm.at[cc, pl.ds(h * HB, HB), :], buf, sem_x)

    def fire_bg(r, c):
        @pl.when(c == -2)
        def _():
            wdesc(cbuf, r, 0, sem_w).start()
            wdesc(cbuf, r, 1, sem_w).start()

        @pl.when(c == -1)
        def _():
            wdesc(zbuf, r, 0, sem_w).start()
            wdesc(zbuf, r, 1, sem_w).start()

    def wait_bg(r, c):
        @pl.when(c == -2)
        def _():
            wdesc(cbuf, r, 0, sem_w).wait()
            wdesc(cbuf, r, 1, sem_w).wait()

        @pl.when(c == -1)
        def _():
            wdesc(zbuf, r, 0, sem_w).wait()
            wdesc(zbuf, r, 1, sem_w).wait()

    def slab_step(k, pxr):
        r = r0 + k
        c = code_s[r]

        @pl.when(k >= LAG)
        def _():
            rp = r - LAG
            wait_bg(rp, code_s[rp])

        fire_bg(r, c)

        @pl.when(c >= 0)
        def _():
            @pl.when(pxr >= 0)
            def _():
                wdesc(xb0, pxr, 0, sem_x).wait()
                wdesc(xb1, pxr, 1, sem_x).wait()
            gdesc(c, 0, xb0).start()
            gdesc(c, 1, xb1).start()
            gdesc(c, 0, xb0).wait()
            gdesc(c, 1, xb1).wait()
            wdesc(xb0, r, 0, sem_x).start()
            wdesc(xb1, r, 1, sem_x).start()
        return jnp.where(c >= 0, r, pxr)

    pxr = lax.fori_loop(0, RSPAN, slab_step, jnp.int32(-1))

    for i in range(LAG):
        rp = r0 + RSPAN - LAG + i
        wait_bg(rp, code_s[rp])

    @pl.when(pxr >= 0)
    def _():
        wdesc(xb0, pxr, 0, sem_x).wait()
        wdesc(xb1, pxr, 1, sem_x).wait()

    @pl.when(wid == 0)
    def _():
        cl = code_s[NUM_PATCHES]

        @pl.when(cl == -2)
        def _():
            pltpu.sync_copy(cbuf, out_hbm.at[NUM_PATCHES, pl.ds(0, HB), :])
            pltpu.sync_copy(cbuf, out_hbm.at[NUM_PATCHES, pl.ds(HB, HB), :])

        @pl.when(cl == -1)
        def _():
            pltpu.sync_copy(zbuf, out_hbm.at[NUM_PATCHES, pl.ds(0, HB), :])
            pltpu.sync_copy(zbuf, out_hbm.at[NUM_PATCHES, pl.ds(HB, HB), :])

        @pl.when(cl >= 0)
        def _():
            pltpu.sync_copy(x_hbm.at[cl, pl.ds(0, HB), :], xb0)
            pltpu.sync_copy(x_hbm.at[cl, pl.ds(HB, HB), :], xb1)
            pltpu.sync_copy(xb0, out_hbm.at[NUM_PATCHES, pl.ds(0, HB), :])
            pltpu.sync_copy(xb1, out_hbm.at[NUM_PATCHES, pl.ds(HB, HB), :])


@functools.partial(jax.jit, static_argnames=())
def kernel(x, sample_index, mask_index, W, b):
    x_t = jnp.transpose(x, (1, 0, 2))
    wrow = W[0].reshape(IN_CHANS * PATCH * PATCH)
    run = pl.kernel(
        _body,
        mesh=plsc.VectorSubcoreMesh(core_axis_name="c", subcore_axis_name="s"),
        out_type=jax.ShapeDtypeStruct((ROWS, B, EMBED), jnp.float32),
        compiler_params=pltpu.CompilerParams(use_tc_tiling_on_sc=True),
        scratch_types=[
            pltpu.VMEM((N_VIS,), jnp.int32),
            pltpu.VMEM((N_MASK,), jnp.int32),
            pltpu.SMEM((ROWS,), jnp.int32),
            pltpu.VMEM((EMBED,), jnp.float32),
            pltpu.VMEM((16,), jnp.float32),
            pltpu.VMEM((HB, EMBED), jnp.float32),
            pltpu.VMEM((HB, EMBED), jnp.float32),
            pltpu.VMEM((HB, EMBED), jnp.float32),
            pltpu.VMEM((HB, EMBED), jnp.float32),
            pltpu.SemaphoreType.DMA,
            pltpu.SemaphoreType.DMA,
            pltpu.SemaphoreType.DMA,
        ],
    )
    out_t = run(x_t, sample_index, mask_index, wrow, b)
    return jnp.transpose(out_t, (1, 0, 2))
